# Initial kernel scaffold; baseline (speedup 1.0000x reference)
import jax
import jax.numpy as jnp
from jax.experimental import pallas as pl


def kernel(x, edge_index, edge_weight, W0, W1, b):
    raise NotImplementedError("write your pallas kernel here")


# speedup vs baseline: 7.4055x; 7.4055x over previous
import functools

import jax
import jax.numpy as jnp
from jax import lax
from jax.experimental import pallas as pl
from jax.experimental.pallas import tpu as pltpu
from jax.experimental.pallas import tpu_sc as plsc

_NC = 2
_NS = 16
_NW = _NC * _NS
_L = 16
_B = 128


def _sc_mesh():
    return plsc.VectorSubcoreMesh(core_axis_name="c", subcore_axis_name="s")


def _deg_call(N, NP, CH):
    rpt = NP // _NS

    @functools.partial(
        pl.kernel,
        out_type=jax.ShapeDtypeStruct((_NC, NP, 128), jnp.float32),
        mesh=_sc_mesh(),
        scratch_types=[
            pltpu.VMEM((_B,), jnp.int32),
            pltpu.VMEM((_B,), jnp.int32),
            pltpu.VMEM((_B,), jnp.float32),
            pltpu.VMEM((_B, 128), jnp.float32),
            pltpu.VMEM_SHARED((NP, 128), jnp.float32),
        ],
    )
    def k(rowp, colp, wp, degp, rb, cb, wb, buf, acc):
        cid = lax.axis_index("c")
        sid = lax.axis_index("s")
        blk = cid * _NS + sid

        def zb(i, _):
            for g in range(128 // _L):
                buf[i, pl.ds(g * _L, _L)] = jnp.zeros((_L,), jnp.float32)
            return 0
        lax.fori_loop(0, _B, zb, 0)
        for t in range(rpt // _B):
            pltpu.sync_copy(buf, acc.at[pl.ds(sid * rpt + t * _B, _B)])
        plsc.subcore_barrier()

        def body(j, _):
            pltpu.sync_copy(rowp.at[blk, j], rb)
            pltpu.sync_copy(colp.at[blk, j], cb)
            pltpu.sync_copy(wp.at[blk, j], wb)

            def grp(g, _):
                sl = pl.ds(g * _L, _L)
                r16 = rb[sl]
                c16 = cb[sl]
                w16 = wb[sl]
                wz = jnp.where(r16 == c16, jnp.float32(0.0), w16)
                for l in range(_L):
                    buf[g * _L + l, pl.ds(0, _L)] = jnp.full(
                        (_L,), wz[l], jnp.float32)
                return 0
            lax.fori_loop(0, _B // _L, grp, 0)
            pltpu.sync_copy(buf, acc.at[rb], add=True)
            return 0
        lax.fori_loop(0, CH, body, 0)
        plsc.subcore_barrier()

        base = sid * rpt
        pltpu.sync_copy(acc.at[pl.ds(base, rpt)],
                        degp.at[cid, pl.ds(base, rpt)])

    return k


def _main_call(N, NP, D, CH):
    rpt = NP // _NS

    @functools.partial(
        pl.kernel,
        out_type=jax.ShapeDtypeStruct((_NC, NP, D), jnp.float32),
        mesh=_sc_mesh(),
        scratch_types=[
            pltpu.VMEM((_B,), jnp.int32),
            pltpu.VMEM((_B,), jnp.int32),
            pltpu.VMEM((_B,), jnp.int32),
            pltpu.VMEM((_B,), jnp.int32),
            pltpu.VMEM((_B,), jnp.float32),
            pltpu.VMEM((_B,), jnp.float32),
            pltpu.VMEM((_B, 128), jnp.float32),
            pltpu.VMEM((_B, 128), jnp.float32),
            pltpu.SemaphoreType.DMA,
            pltpu.SemaphoreType.DMA,
            pltpu.VMEM_SHARED((NP, 128), jnp.float32),
        ],
    )
    def k(rowp, colp, wp, ys, accp, rb0, rb1, cb0, cb1, wb0, wb1,
          buf0, buf1, sem0, sem1, acc):
        cid = lax.axis_index("c")
        sid = lax.axis_index("s")
        blk = cid * _NS + sid

        def zb(i, _):
            for g in range(128 // _L):
                buf0[i, pl.ds(g * _L, _L)] = jnp.zeros((_L,), jnp.float32)
            return 0
        lax.fori_loop(0, _B, zb, 0)
        for t in range(rpt // _B):
            pltpu.sync_copy(buf0, acc.at[pl.ds(sid * rpt + t * _B, _B)])
        plsc.subcore_barrier()

        def scale(buf, rb, cb, wb):
            def grp(g, _):
                sl = pl.ds(g * _L, _L)
                r16 = rb[sl]
                c16 = cb[sl]
                w16 = wb[sl]
                cv = jnp.where(r16 == c16, jnp.float32(0.0), -w16)
                for l in range(_L):
                    cf = jnp.full((_L,), cv[l], jnp.float32)
                    r = g * _L + l
                    for k8 in range(128 // _L):
                        s2 = pl.ds(k8 * _L, _L)
                        buf[r, s2] = buf[r, s2] * cf
                return 0
            lax.fori_loop(0, _B // _L, grp, 0)

        def body(j2, _):
            ja = j2 * 2
            jb = ja + 1
            pltpu.sync_copy(rowp.at[blk, ja], rb0)
            pltpu.sync_copy(rowp.at[blk, jb], rb1)
            pltpu.sync_copy(colp.at[blk, ja], cb0)
            pltpu.sync_copy(colp.at[blk, jb], cb1)
            pltpu.sync_copy(wp.at[blk, ja], wb0)
            pltpu.sync_copy(wp.at[blk, jb], wb1)
            ca = pltpu.async_copy(ys.at[rb0], buf0, sem0)
            cb_ = pltpu.async_copy(ys.at[rb1], buf1, sem1)
            ca.wait()
            scale(buf0, rb0, cb0, wb0)
            pltpu.sync_copy(buf0, acc.at[cb0], add=True)
            cb_.wait()
            scale(buf1, rb1, cb1, wb1)
            pltpu.sync_copy(buf1, acc.at[cb1], add=True)
            return 0
        lax.fori_loop(0, CH // 2, body, 0)
        plsc.subcore_barrier()

        base = sid * rpt
        pltpu.sync_copy(acc.at[pl.ds(base, rpt)],
                        accp.at[cid, pl.ds(base, rpt)])

    return k


def _pre_body(degp_ref, x_ref, w0_ref, w1_ref, b_ref, ys_ref, z_ref):
    deg = jnp.sum(degp_ref[...], axis=(0, 2)) * jnp.float32(1.0 / _L)
    pos = deg > 0
    dis = jnp.where(pos, lax.rsqrt(jnp.where(pos, deg, 1.0)), 0.0)
    xb = x_ref[...]
    ys_ref[...] = dis[:, None] * jnp.dot(
        xb, w1_ref[...], preferred_element_type=jnp.float32)
    z_ref[...] = jnp.dot(
        xb, w0_ref[...], preferred_element_type=jnp.float32) + b_ref[...][None, :]


def _post_body(degp_ref, accp_ref, z_ref, out_ref):
    deg = jnp.sum(degp_ref[...], axis=(0, 2)) * jnp.float32(1.0 / _L)
    pos = deg > 0
    dis = jnp.where(pos, lax.rsqrt(jnp.where(pos, deg, 1.0)), 0.0)
    a = jnp.sum(accp_ref[...], axis=0)
    out_ref[...] = jnp.maximum(z_ref[...] + dis[:, None] * a, 0.0)


def kernel(x, edge_index, edge_weight, W0, W1, b):
    N, D = x.shape
    E = edge_weight.shape[0]
    CH = -(-E // (_NW * _B))
    if CH % 2:
        CH += 1
    epad = _NW * CH * _B
    NP = -(-N // (_NS * _B)) * (_NS * _B)
    row = edge_index[0]
    col = edge_index[1]
    pad = epad - E
    rowp = jnp.concatenate([row, jnp.zeros((pad,), jnp.int32)]).reshape(_NW, CH, _B)
    colp = jnp.concatenate([col, jnp.zeros((pad,), jnp.int32)]).reshape(_NW, CH, _B)
    wp = jnp.concatenate(
        [edge_weight, jnp.zeros((pad,), jnp.float32)]).reshape(_NW, CH, _B)

    degp = _deg_call(N, NP, CH)(rowp, colp, wp)

    RB = 1000
    ys, z = pl.pallas_call(
        _pre_body,
        grid=(N // RB,),
        in_specs=[
            pl.BlockSpec((_NC, RB, 128), lambda i: (0, i, 0)),
            pl.BlockSpec((RB, D), lambda i: (i, 0)),
            pl.BlockSpec((D, D), lambda i: (0, 0)),
            pl.BlockSpec((D, D), lambda i: (0, 0)),
            pl.BlockSpec((D,), lambda i: (0,)),
        ],
        out_specs=[
            pl.BlockSpec((RB, D), lambda i: (i, 0)),
            pl.BlockSpec((RB, D), lambda i: (i, 0)),
        ],
        out_shape=[
            jax.ShapeDtypeStruct((N, D), jnp.float32),
            jax.ShapeDtypeStruct((N, D), jnp.float32),
        ],
    )(degp, x, W0, W1, b)

    accp = _main_call(N, NP, D, CH)(rowp, colp, wp, ys)

    out = pl.pallas_call(
        _post_body,
        grid=(N // RB,),
        in_specs=[
            pl.BlockSpec((_NC, RB, 128), lambda i: (0, i, 0)),
            pl.BlockSpec((_NC, RB, D), lambda i: (0, i, 0)),
            pl.BlockSpec((RB, D), lambda i: (i, 0)),
        ],
        out_specs=pl.BlockSpec((RB, D), lambda i: (i, 0)),
        out_shape=jax.ShapeDtypeStruct((N, D), jnp.float32),
    )(degp, accp, z)
    return out


# speedup vs baseline: 9.4806x; 1.2802x over previous
import functools

import jax
import jax.numpy as jnp
from jax import lax
from jax.experimental import pallas as pl
from jax.experimental.pallas import tpu as pltpu
from jax.experimental.pallas import tpu_sc as plsc

_NC = 2
_NS = 16
_NW = _NC * _NS
_L = 16
_B = 128


def _sc_mesh():
    return plsc.VectorSubcoreMesh(core_axis_name="c", subcore_axis_name="s")


def _deg_call(N, NP, CH):
    rpt = NP // _NS

    @functools.partial(
        pl.kernel,
        out_type=jax.ShapeDtypeStruct((_NC, NP, 128), jnp.float32),
        mesh=_sc_mesh(),
        scratch_types=[
            pltpu.VMEM((_B,), jnp.int32),
            pltpu.VMEM((_B,), jnp.int32),
            pltpu.VMEM((_B,), jnp.float32),
            pltpu.VMEM((_B, 128), jnp.float32),
            pltpu.VMEM_SHARED((NP, 128), jnp.float32),
        ],
    )
    def k(rowp, colp, wp, degp, rb, cb, wb, buf, acc):
        cid = lax.axis_index("c")
        sid = lax.axis_index("s")
        blk = cid * _NS + sid

        def zb(i, _):
            for g in range(128 // _L):
                buf[i, pl.ds(g * _L, _L)] = jnp.zeros((_L,), jnp.float32)
            return 0
        lax.fori_loop(0, _B, zb, 0)
        for t in range(rpt // _B):
            pltpu.sync_copy(buf, acc.at[pl.ds(sid * rpt + t * _B, _B)])
        plsc.subcore_barrier()

        def body(j, _):
            pltpu.sync_copy(rowp.at[blk, j], rb)
            pltpu.sync_copy(colp.at[blk, j], cb)
            pltpu.sync_copy(wp.at[blk, j], wb)

            def grp(g, _):
                sl = pl.ds(g * _L, _L)
                r16 = rb[sl]
                c16 = cb[sl]
                w16 = wb[sl]
                wz = jnp.where(r16 == c16, jnp.float32(0.0), w16)
                for l in range(_L):
                    buf[g * _L + l, pl.ds(0, _L)] = jnp.full(
                        (_L,), wz[l], jnp.float32)
                return 0
            lax.fori_loop(0, _B // _L, grp, 0)
            pltpu.sync_copy(buf, acc.at[rb], add=True)
            return 0
        lax.fori_loop(0, CH, body, 0)
        plsc.subcore_barrier()

        base = sid * rpt
        pltpu.sync_copy(acc.at[pl.ds(base, rpt)],
                        degp.at[cid, pl.ds(base, rpt)])

    return k


def _main_call(N, NP, D, CH):
    rpt = NP // _NS
    Q = 8
    NQ = CH // Q

    @functools.partial(
        pl.kernel,
        out_type=jax.ShapeDtypeStruct((_NC, NP, D), jnp.float32),
        mesh=_sc_mesh(),
        scratch_types=[
            pltpu.VMEM((3, Q, _B), jnp.int32),
            pltpu.VMEM((3, Q, _B), jnp.int32),
            pltpu.VMEM((3, Q, _B), jnp.float32),
            pltpu.VMEM((_B, 128), jnp.float32),
            pltpu.VMEM((_B, 128), jnp.float32),
            pltpu.SemaphoreType.DMA,
            pltpu.SemaphoreType.DMA,
            pltpu.SemaphoreType.DMA,
            pltpu.SemaphoreType.DMA,
            pltpu.VMEM_SHARED((NP, 128), jnp.float32),
        ],
    )
    def k(rowp, colp, wp, ys, accp, rbm, cbm, wbm, buf0, buf1,
          gsem0, gsem1, ssem0, ssem1, acc):
        cid = lax.axis_index("c")
        sid = lax.axis_index("s")
        blk = cid * _NS + sid

        def zb(i, _):
            for g in range(128 // _L):
                buf0[i, pl.ds(g * _L, _L)] = jnp.zeros((_L,), jnp.float32)
            return 0
        lax.fori_loop(0, _B, zb, 0)
        for t in range(rpt // _B):
            pltpu.sync_copy(buf0, acc.at[pl.ds(sid * rpt + t * _B, _B)])
        plsc.subcore_barrier()

        def scale(buf, slot, t):
            def grp(g, _):
                sl = pl.ds(g * _L, _L)
                r16 = rbm[slot, t, sl]
                c16 = cbm[slot, t, sl]
                w16 = wbm[slot, t, sl]
                cv = jnp.where(r16 == c16, jnp.float32(0.0), -w16)
                for l in range(_L):
                    cf = jnp.full((_L,), cv[l], jnp.float32)
                    r = g * _L + l
                    for k8 in range(128 // _L):
                        s2 = pl.ds(k8 * _L, _L)
                        buf[r, s2] = buf[r, s2] * cf
                return 0
            lax.fori_loop(0, _B // _L, grp, 0)

        def stage(q, slot):
            pltpu.sync_copy(rowp.at[blk, pl.ds(q * Q, Q)], rbm.at[slot])
            pltpu.sync_copy(colp.at[blk, pl.ds(q * Q, Q)], cbm.at[slot])
            pltpu.sync_copy(wp.at[blk, pl.ds(q * Q, Q)], wbm.at[slot])

        stage(0, 0)
        stage(1, 1)
        pltpu.async_copy(ys.at[rbm.at[0, 0]], buf0, gsem0)

        bufs = (buf0, buf1)
        gsems = (gsem0, gsem1)
        ssems = (ssem0, ssem1)

        def pair(j2, _):
            for p in range(2):
                j = j2 * 2 + p
                q = j // Q
                t = j % Q
                slot = q % 3

                @pl.when(jnp.logical_and(t == 0, q + 2 <= NQ - 1))
                def _():
                    stage(q + 2, (q + 2) % 3)

                @pl.when(j + 1 <= CH - 1)
                def _():
                    j1 = j + 1
                    pltpu.async_copy(
                        ys.at[rbm.at[(j1 // Q) % 3, j1 % Q]],
                        bufs[1 - p], gsems[1 - p])

                pltpu.make_async_copy(
                    ys.at[rbm.at[slot, t]], bufs[p], gsems[p]).wait()
                scale(bufs[p], slot, t)
                pltpu.sync_copy(bufs[p], acc.at[cbm.at[slot, t]], add=True)
            return 0
        lax.fori_loop(0, CH // 2, pair, 0)
        plsc.subcore_barrier()

        base = sid * rpt
        pltpu.sync_copy(acc.at[pl.ds(base, rpt)],
                        accp.at[cid, pl.ds(base, rpt)])

    return k


def _pre_body(degp_ref, x_ref, w0_ref, w1_ref, b_ref, ys_ref, z_ref):
    deg = jnp.sum(degp_ref[...], axis=(0, 2)) * jnp.float32(1.0 / _L)
    pos = deg > 0
    dis = jnp.where(pos, lax.rsqrt(jnp.where(pos, deg, 1.0)), 0.0)
    xb = x_ref[...]
    ys_ref[...] = dis[:, None] * jnp.dot(
        xb, w1_ref[...], preferred_element_type=jnp.float32)
    z_ref[...] = jnp.dot(
        xb, w0_ref[...], preferred_element_type=jnp.float32) + b_ref[...][None, :]


def _post_body(degp_ref, accp_ref, z_ref, out_ref):
    deg = jnp.sum(degp_ref[...], axis=(0, 2)) * jnp.float32(1.0 / _L)
    pos = deg > 0
    dis = jnp.where(pos, lax.rsqrt(jnp.where(pos, deg, 1.0)), 0.0)
    a = jnp.sum(accp_ref[...], axis=0)
    out_ref[...] = jnp.maximum(z_ref[...] + dis[:, None] * a, 0.0)


def kernel(x, edge_index, edge_weight, W0, W1, b):
    N, D = x.shape
    E = edge_weight.shape[0]
    CH = -(-E // (_NW * _B))
    if CH % 2:
        CH += 1
    epad = _NW * CH * _B
    NP = -(-N // (_NS * _B)) * (_NS * _B)
    row = edge_index[0]
    col = edge_index[1]
    pad = epad - E
    rowp = jnp.concatenate([row, jnp.zeros((pad,), jnp.int32)]).reshape(_NW, CH, _B)
    colp = jnp.concatenate([col, jnp.zeros((pad,), jnp.int32)]).reshape(_NW, CH, _B)
    wp = jnp.concatenate(
        [edge_weight, jnp.zeros((pad,), jnp.float32)]).reshape(_NW, CH, _B)

    degp = _deg_call(N, NP, CH)(rowp, colp, wp)

    RB = 1000
    ys, z = pl.pallas_call(
        _pre_body,
        grid=(N // RB,),
        in_specs=[
            pl.BlockSpec((_NC, RB, 128), lambda i: (0, i, 0)),
            pl.BlockSpec((RB, D), lambda i: (i, 0)),
            pl.BlockSpec((D, D), lambda i: (0, 0)),
            pl.BlockSpec((D, D), lambda i: (0, 0)),
            pl.BlockSpec((D,), lambda i: (0,)),
        ],
        out_specs=[
            pl.BlockSpec((RB, D), lambda i: (i, 0)),
            pl.BlockSpec((RB, D), lambda i: (i, 0)),
        ],
        out_shape=[
            jax.ShapeDtypeStruct((N, D), jnp.float32),
            jax.ShapeDtypeStruct((N, D), jnp.float32),
        ],
    )(degp, x, W0, W1, b)

    accp = _main_call(N, NP, D, CH)(rowp, colp, wp, ys)

    out = pl.pallas_call(
        _post_body,
        grid=(N // RB,),
        in_specs=[
            pl.BlockSpec((_NC, RB, 128), lambda i: (0, i, 0)),
            pl.BlockSpec((_NC, RB, D), lambda i: (0, i, 0)),
            pl.BlockSpec((RB, D), lambda i: (i, 0)),
        ],
        out_specs=pl.BlockSpec((RB, D), lambda i: (i, 0)),
        out_shape=jax.ShapeDtypeStruct((N, D), jnp.float32),
    )(degp, accp, z)
    return out


# speedup vs baseline: 11.2627x; 1.1880x over previous
import functools

import jax
import jax.numpy as jnp
from jax import lax
from jax.experimental import pallas as pl
from jax.experimental.pallas import tpu as pltpu
from jax.experimental.pallas import tpu_sc as plsc

_NC = 2
_NS = 16
_NW = _NC * _NS
_L = 16
_B = 128


def _sc_mesh():
    return plsc.VectorSubcoreMesh(core_axis_name="c", subcore_axis_name="s")


def _deg_call(N, NP, CH):
    rpt = NP // _NS

    @functools.partial(
        pl.kernel,
        out_type=jax.ShapeDtypeStruct((_NC, NP, 128), jnp.float32),
        mesh=_sc_mesh(),
        scratch_types=[
            pltpu.VMEM((8, _B), jnp.int32),
            pltpu.VMEM((8, _B), jnp.int32),
            pltpu.VMEM((8, _B), jnp.float32),
            pltpu.VMEM((_B, 128), jnp.float32),
            pltpu.VMEM_SHARED((NP, 128), jnp.float32),
        ],
    )
    def k(rowp, colp, wp, degp, rb, cb, wb, buf, acc):
        cid = lax.axis_index("c")
        sid = lax.axis_index("s")
        blk = cid * _NS + sid

        def zb(i, _):
            for g in range(128 // _L):
                buf[i, pl.ds(g * _L, _L)] = jnp.zeros((_L,), jnp.float32)
            return 0
        lax.fori_loop(0, _B, zb, 0)
        for t in range(rpt // _B):
            pltpu.sync_copy(buf, acc.at[pl.ds(sid * rpt + t * _B, _B)])
        plsc.subcore_barrier()

        QA = 8
        def body(q, _):
            pltpu.sync_copy(rowp.at[blk, pl.ds(q * QA, QA)], rb)
            pltpu.sync_copy(colp.at[blk, pl.ds(q * QA, QA)], cb)
            pltpu.sync_copy(wp.at[blk, pl.ds(q * QA, QA)], wb)
            def chunk(t, _):
                def grp(g, _):
                    sl = pl.ds(g * _L, _L)
                    r16 = rb[t, sl]
                    c16 = cb[t, sl]
                    w16 = wb[t, sl]
                    wz = jnp.where(r16 == c16, jnp.float32(0.0), w16)
                    for l in range(_L):
                        buf[g * _L + l, pl.ds(0, _L)] = jnp.full(
                            (_L,), wz[l], jnp.float32)
                    return 0
                lax.fori_loop(0, _B // _L, grp, 0)
                pltpu.sync_copy(buf, acc.at[rb.at[t]], add=True)
                return 0
            lax.fori_loop(0, QA, chunk, 0)
            return 0
        lax.fori_loop(0, CH // QA, body, 0)
        plsc.subcore_barrier()

        base = sid * rpt
        pltpu.sync_copy(acc.at[pl.ds(base, rpt)],
                        degp.at[cid, pl.ds(base, rpt)])

    return k


def _main_call(N, NP, D, CH):
    rpt = NP // _NS
    Q = 8
    NQ = CH // Q

    @functools.partial(
        pl.kernel,
        out_type=jax.ShapeDtypeStruct((_NC, NP, D), jnp.float32),
        mesh=_sc_mesh(),
        scratch_types=[
            pltpu.VMEM((3, Q, _B), jnp.int32),
            pltpu.VMEM((3, Q, _B), jnp.int32),
            pltpu.VMEM((3, Q, _B), jnp.float32),
            pltpu.VMEM((_B, 128), jnp.float32),
            pltpu.VMEM((_B, 128), jnp.float32),
            pltpu.SemaphoreType.DMA,
            pltpu.SemaphoreType.DMA,
            pltpu.SemaphoreType.DMA,
            pltpu.SemaphoreType.DMA,
            pltpu.VMEM_SHARED((NP, 128), jnp.float32),
        ],
    )
    def k(rowp, colp, wp, ys, accp, rbm, cbm, wbm, buf0, buf1,
          gsem0, gsem1, ssem0, ssem1, acc):
        cid = lax.axis_index("c")
        sid = lax.axis_index("s")
        blk = cid * _NS + sid

        def zb(i, _):
            for g in range(128 // _L):
                buf0[i, pl.ds(g * _L, _L)] = jnp.zeros((_L,), jnp.float32)
            return 0
        lax.fori_loop(0, _B, zb, 0)
        for t in range(rpt // _B):
            pltpu.sync_copy(buf0, acc.at[pl.ds(sid * rpt + t * _B, _B)])
        plsc.subcore_barrier()

        def scale(buf, slot, t):
            def grp(g2, _):
                for u in range(2):
                    g = g2 * 2 + u
                    sl = pl.ds(g * _L, _L)
                    r16 = rbm[slot, t, sl]
                    c16 = cbm[slot, t, sl]
                    w16 = wbm[slot, t, sl]
                    cv = jnp.where(r16 == c16, jnp.float32(0.0), -w16)
                    for l in range(_L):
                        cf = jnp.full((_L,), cv[l], jnp.float32)
                        r = g * _L + l
                        for k8 in range(128 // _L):
                            s2 = pl.ds(k8 * _L, _L)
                            buf[r, s2] = buf[r, s2] * cf
                return 0
            lax.fori_loop(0, _B // (2 * _L), grp, 0)

        def stage(q, slot):
            pltpu.sync_copy(rowp.at[blk, pl.ds(q * Q, Q)], rbm.at[slot])
            pltpu.sync_copy(colp.at[blk, pl.ds(q * Q, Q)], cbm.at[slot])
            pltpu.sync_copy(wp.at[blk, pl.ds(q * Q, Q)], wbm.at[slot])

        stage(0, 0)
        stage(1, 1)
        pltpu.async_copy(ys.at[rbm.at[0, 0]], buf0, gsem0)

        bufs = (buf0, buf1)
        gsems = (gsem0, gsem1)
        ssems = (ssem0, ssem1)

        def pair(j2, _):
            for p in range(2):
                j = j2 * 2 + p
                q = j // Q
                t = j % Q
                slot = q % 3

                @pl.when(jnp.logical_and(t == 0, q + 2 <= NQ - 1))
                def _():
                    stage(q + 2, (q + 2) % 3)

                @pl.when(j + 1 <= CH - 1)
                def _():
                    j1 = j + 1
                    pltpu.async_copy(
                        ys.at[rbm.at[(j1 // Q) % 3, j1 % Q]],
                        bufs[1 - p], gsems[1 - p])

                pltpu.make_async_copy(
                    ys.at[rbm.at[slot, t]], bufs[p], gsems[p]).wait()
                scale(bufs[p], slot, t)
                pltpu.sync_copy(bufs[p], acc.at[cbm.at[slot, t]], add=True)
            return 0
        lax.fori_loop(0, CH // 2, pair, 0)
        plsc.subcore_barrier()

        base = sid * rpt
        pltpu.sync_copy(acc.at[pl.ds(base, rpt)],
                        accp.at[cid, pl.ds(base, rpt)])

    return k


def _pre_body(degp_ref, x_ref, w0_ref, w1_ref, b_ref, ys_ref, z_ref):
    deg = jnp.sum(degp_ref[...], axis=(0, 2)) * jnp.float32(1.0 / _L)
    pos = deg > 0
    dis = jnp.where(pos, lax.rsqrt(jnp.where(pos, deg, 1.0)), 0.0)
    xb = x_ref[...]
    ys_ref[...] = dis[:, None] * jnp.dot(
        xb, w1_ref[...], preferred_element_type=jnp.float32)
    z_ref[...] = jnp.dot(
        xb, w0_ref[...], preferred_element_type=jnp.float32) + b_ref[...][None, :]


def _post_body(degp_ref, accp_ref, z_ref, out_ref):
    deg = jnp.sum(degp_ref[...], axis=(0, 2)) * jnp.float32(1.0 / _L)
    pos = deg > 0
    dis = jnp.where(pos, lax.rsqrt(jnp.where(pos, deg, 1.0)), 0.0)
    a = jnp.sum(accp_ref[...], axis=0)
    out_ref[...] = jnp.maximum(z_ref[...] + dis[:, None] * a, 0.0)


def kernel(x, edge_index, edge_weight, W0, W1, b):
    N, D = x.shape
    E = edge_weight.shape[0]
    CH = -(-E // (_NW * _B))
    if CH % 2:
        CH += 1
    epad = _NW * CH * _B
    NP = -(-N // (_NS * _B)) * (_NS * _B)
    row = edge_index[0]
    col = edge_index[1]
    pad = epad - E
    rowp = jnp.concatenate([row, jnp.zeros((pad,), jnp.int32)]).reshape(_NW, CH, _B)
    colp = jnp.concatenate([col, jnp.zeros((pad,), jnp.int32)]).reshape(_NW, CH, _B)
    wp = jnp.concatenate(
        [edge_weight, jnp.zeros((pad,), jnp.float32)]).reshape(_NW, CH, _B)

    degp = _deg_call(N, NP, CH)(rowp, colp, wp)

    RB = 1000
    ys, z = pl.pallas_call(
        _pre_body,
        grid=(N // RB,),
        in_specs=[
            pl.BlockSpec((_NC, RB, 128), lambda i: (0, i, 0)),
            pl.BlockSpec((RB, D), lambda i: (i, 0)),
            pl.BlockSpec((D, D), lambda i: (0, 0)),
            pl.BlockSpec((D, D), lambda i: (0, 0)),
            pl.BlockSpec((D,), lambda i: (0,)),
        ],
        out_specs=[
            pl.BlockSpec((RB, D), lambda i: (i, 0)),
            pl.BlockSpec((RB, D), lambda i: (i, 0)),
        ],
        out_shape=[
            jax.ShapeDtypeStruct((N, D), jnp.float32),
            jax.ShapeDtypeStruct((N, D), jnp.float32),
        ],
    )(degp, x, W0, W1, b)

    accp = _main_call(N, NP, D, CH)(rowp, colp, wp, ys)

    out = pl.pallas_call(
        _post_body,
        grid=(N // RB,),
        in_specs=[
            pl.BlockSpec((_NC, RB, 128), lambda i: (0, i, 0)),
            pl.BlockSpec((_NC, RB, D), lambda i: (0, i, 0)),
            pl.BlockSpec((RB, D), lambda i: (i, 0)),
        ],
        out_specs=pl.BlockSpec((RB, D), lambda i: (i, 0)),
        out_shape=jax.ShapeDtypeStruct((N, D), jnp.float32),
    )(degp, accp, z)
    return out
